# baseline (speedup 1.0000x reference)
import functools

import jax
import jax.numpy as jnp
from jax import lax
from jax.experimental import pallas as pl
from jax.experimental.pallas import tpu as pltpu
from jax.experimental.pallas import tpu_sc as plsc

B, S, A = 16, 64, 10
VOCAB, EMB, HID = 100000, 300, 64
EMBP = 384
NH_ATT = 32
HD_ATT = HID // NH_ATT
H_GAT = 16
POL = 3
F32 = jnp.float32


def _lrelu(x, slope):
    return jnp.where(x >= 0, x, x * slope)


_NROWS = S * B + A * B
_NPAD = 1280
_NW = 32
_RPW = _NPAD // _NW


def _sc_gather(table, idx):
    mesh = plsc.VectorSubcoreMesh(core_axis_name="c", subcore_axis_name="s")

    @functools.partial(
        pl.kernel,
        mesh=mesh,
        out_type=jax.ShapeDtypeStruct((_NPAD, EMBP), F32),
        scratch_types=[
            pltpu.VMEM((_RPW,), jnp.int32),
            pltpu.VMEM((_RPW, EMBP), F32),
            pltpu.SemaphoreType.DMA,
        ],
    )
    def gather_k(table_hbm, idx_hbm, out_hbm, idx_v, rows_v, sem):
        wid = lax.axis_index("s") * 2 + lax.axis_index("c")
        base = wid * _RPW
        pltpu.sync_copy(idx_hbm.at[pl.ds(base, _RPW)], idx_v)
        pltpu.async_copy(table_hbm.at[idx_v], rows_v, sem).wait()
        pltpu.sync_copy(rows_v, out_hbm.at[pl.ds(base, _RPW)])

    return gather_k(table, idx)


def _gru_body(ti_ref, ai_ref, li_ref, adj_ref, xc_ref, xa_ref,
              wihc_ref, whhc_ref, bihc_ref, bhhc_ref,
              wiha_ref, whha_ref, biha_ref, bhha_ref,
              ctx_out_ref, asp_out_ref, base_ref, maskp_ref, tl_ref, al_ref,
              xpc_ref, xpa_ref):
    tl = jnp.sum((ti_ref[...] != 0).astype(F32), axis=1, keepdims=True) + 5.0
    al = jnp.sum((ai_ref[...] != 0).astype(F32), axis=1, keepdims=True)
    ll = jnp.sum((li_ref[...] != 0).astype(F32), axis=1, keepdims=True)
    tl_ref[...] = tl
    al_ref[...] = al

    j = lax.broadcasted_iota(jnp.int32, (B, S), 1).astype(F32)
    base = (j >= ll) & (j <= ll + al - 1.0)
    adj_sum = jnp.sum(adj_ref[...][:, :, S - 5:], axis=2)
    maskp = base | ((adj_sum != 0) & (j < S - 5))
    base_ref[...] = base.astype(F32)
    maskp_ref[...] = maskp.astype(F32)

    def run_gru(x2d, wih_t, whh_t, bih, bhh, T, lens, out_ref, xp_ref):
        xp = jnp.dot(x2d, wih_t, preferred_element_type=F32) + bih
        xp_ref[...] = xp.reshape(B, T, 3 * HID)

        def step(t, h):
            xpt = xp_ref[:, pl.ds(t, 1), :][:, 0, :]
            hh = jnp.dot(h, whh_t, preferred_element_type=F32)
            r = jax.nn.sigmoid(xpt[:, 0:HID] + hh[:, 0:HID] + bhh[:, 0:HID])
            z = jax.nn.sigmoid(xpt[:, HID:2 * HID] + hh[:, HID:2 * HID]
                               + bhh[:, HID:2 * HID])
            n = jnp.tanh(xpt[:, 2 * HID:] + r * (hh[:, 2 * HID:]
                                                 + bhh[:, 2 * HID:]))
            h2 = (1.0 - z) * n + z * h
            tmask = (lens > t.astype(F32)).astype(F32)
            out_ref[:, pl.ds(t, 1), :] = (h2 * tmask)[:, None, :]
            return h2

        lax.fori_loop(0, T, step, jnp.zeros((B, HID), F32))

    run_gru(xc_ref[...], wihc_ref[...], whhc_ref[...], bihc_ref[...],
            bhhc_ref[...], S, tl, ctx_out_ref, xpc_ref)
    run_gru(xa_ref[...], wiha_ref[...], whha_ref[...], biha_ref[...],
            bhha_ref[...], A, al, asp_out_ref, xpa_ref)


def _gat_body(x_ref, adj_ref, wl_ref, bl_ref, wr_ref, br_ref, att_ref,
              gbias_ref, out_ref):
    x = x_ref[0]
    adjb = adj_ref[0]
    xl = jnp.dot(x, wl_ref[...], preferred_element_type=F32) + bl_ref[...]
    xr = jnp.dot(x, wr_ref[...], preferred_element_type=F32) + br_ref[...]

    r_i = lax.broadcasted_iota(jnp.int32, (S, S), 0)
    c_i = lax.broadcasted_iota(jnp.int32, (S, S), 1)
    allowed = ((adjb != 0) & (r_i != c_i)) | (r_i == c_i)

    acc = jnp.zeros((S, HID), F32)
    for h in range(H_GAT):
        xlh = xl[:, h * HID:(h + 1) * HID]
        xrh = xr[:, h * HID:(h + 1) * HID]
        e = _lrelu(xlh[:, None, :] + xrh[None, :, :], 0.2)
        att3 = att_ref[h:h + 1, :][None]
        logits = jnp.sum(e * att3, axis=2)
        ml = jnp.where(allowed, logits, -1e30)
        amax = jnp.max(ml, axis=0, keepdims=True)
        p = jnp.where(allowed, jnp.exp(logits - amax), 0.0)
        denom = jnp.sum(p, axis=0, keepdims=True)
        alpha = p / (denom + 1e-16)
        acc = acc + lax.dot_general(alpha, xlh, (((0,), (0,)), ((), ())),
                                    preferred_element_type=F32)

    xi = _lrelu(acc * (1.0 / H_GAT) + gbias_ref[...], 0.01)
    out_ref[0] = xi


def _att_head(k, q, wk_ref, bk_ref, wq_ref, bq_ref, blwk_ref, pw_ref, pb_ref,
              epairs, edup):
    kx = jnp.dot(k, wk_ref[...], preferred_element_type=F32) + bk_ref[...]
    qx = jnp.dot(q, wq_ref[...], preferred_element_type=F32) + bq_ref[...]
    kxp = jnp.dot(kx, blwk_ref[...], preferred_element_type=F32)
    logits = jnp.dot(kxp * qx, epairs, preferred_element_type=F32)
    m = jnp.max(logits, axis=0, keepdims=True)
    sc = jnp.exp(logits - m)
    sc = sc / jnp.sum(sc, axis=0, keepdims=True)
    sd = jnp.dot(sc, edup, preferred_element_type=F32)
    outf = jnp.sum(kx * sd, axis=0, keepdims=True)
    return jnp.dot(outf, pw_ref[...], preferred_element_type=F32) + pb_ref[...]


def _final_body(xi_ref, ctx_ref, asp_ref, base_ref, maskp_ref, tl_ref, al_ref,
                aawk_ref, aabk_ref, aawq_ref, aabq_ref, aablwk_ref, aapw_ref,
                aapb_ref, acwk_ref, acbk_ref, acwq_ref, acbq_ref, acblwk_ref,
                acpw_ref, acpb_ref, fcw_ref, fcb_ref, out_ref):
    ip_r = lax.broadcasted_iota(jnp.int32, (HID, NH_ATT), 0)
    ip_c = lax.broadcasted_iota(jnp.int32, (HID, NH_ATT), 1)
    epairs = (ip_r // HD_ATT == ip_c).astype(F32)
    id_r = lax.broadcasted_iota(jnp.int32, (NH_ATT, HID), 0)
    id_c = lax.broadcasted_iota(jnp.int32, (NH_ATT, HID), 1)
    edup = (id_c // HD_ATT == id_r).astype(F32)

    xi = xi_ref[0]
    ctx = ctx_ref[0]
    asp = asp_ref[0]
    x = xi * base_ref[0]
    cm = ctx * maskp_ref[0]

    am = lax.dot_general(x, cm, (((1,), (1,)), ((), ())),
                         preferred_element_type=F32)
    s = jnp.sum(am, axis=0, keepdims=True)
    s = s - jnp.max(s, axis=1, keepdims=True)
    es = jnp.exp(s)
    alpha = es / jnp.sum(es, axis=1, keepdims=True)
    gta = jnp.dot(alpha, cm, preferred_element_type=F32)

    al = al_ref[0]
    tl = tl_ref[0]
    asp_pool = jnp.sum(asp, axis=0, keepdims=True) / al
    ctx_pool = jnp.sum(ctx, axis=0, keepdims=True) / tl

    af = _att_head(asp, ctx_pool, aawk_ref, aabk_ref, aawq_ref, aabq_ref,
                   aablwk_ref, aapw_ref, aapb_ref, epairs, edup)
    cf = _att_head(ctx, asp_pool, acwk_ref, acbk_ref, acwq_ref, acbq_ref,
                   acblwk_ref, acpw_ref, acpb_ref, epairs, edup)

    fcw = fcw_ref[...]
    out = (jnp.dot(af, fcw[0:HID], preferred_element_type=F32)
           + jnp.dot(cf, fcw[HID:2 * HID], preferred_element_type=F32)
           + jnp.dot(gta, fcw[2 * HID:], preferred_element_type=F32)
           + fcb_ref[...])
    out_ref[0] = out


def _rep(shape):
    nd = len(shape)
    return pl.BlockSpec(shape, lambda b: (0,) * nd)


@jax.jit
def kernel(text_indices, aspect_indices, left_indices, adj, embed_table,
           gru_ctx_wih, gru_ctx_whh, gru_ctx_bih, gru_ctx_bhh,
           gru_asp_wih, gru_asp_whh, gru_asp_bih, gru_asp_bhh,
           aa_wk, aa_bk, aa_wq, aa_bq, aa_blw, aa_pw, aa_pb,
           ac_wk, ac_bk, ac_wq, ac_bq, ac_blw, ac_pw, ac_pb,
           gat_wl, gat_bl, gat_wr, gat_br, gat_att, gat_bias, fc_w, fc_b):
    idx_all = jnp.concatenate([
        text_indices.reshape(-1), aspect_indices.reshape(-1),
        jnp.zeros((_NPAD - _NROWS,), jnp.int32)])
    table_p = jnp.pad(embed_table, ((0, 0), (0, EMBP - EMB)))
    rows = _sc_gather(table_p, idx_all)
    xc = rows[:S * B]
    xa = rows[S * B:S * B + A * B]

    ctx, asp, base, maskp, tl, al = pl.pallas_call(
        _gru_body,
        out_shape=[
            jax.ShapeDtypeStruct((B, S, HID), F32),
            jax.ShapeDtypeStruct((B, A, HID), F32),
            jax.ShapeDtypeStruct((B, S), F32),
            jax.ShapeDtypeStruct((B, S), F32),
            jax.ShapeDtypeStruct((B, 1), F32),
            jax.ShapeDtypeStruct((B, 1), F32),
        ],
        scratch_shapes=[
            pltpu.VMEM((B, S, 3 * HID), F32),
            pltpu.VMEM((B, A, 3 * HID), F32),
        ],
    )(text_indices, aspect_indices, left_indices, adj, xc, xa,
      jnp.pad(gru_ctx_wih.T, ((0, EMBP - EMB), (0, 0))), gru_ctx_whh.T,
      gru_ctx_bih[None, :], gru_ctx_bhh[None, :],
      jnp.pad(gru_asp_wih.T, ((0, EMBP - EMB), (0, 0))), gru_asp_whh.T,
      gru_asp_bih[None, :], gru_asp_bhh[None, :])

    xi = pl.pallas_call(
        _gat_body,
        grid=(B,),
        in_specs=[
            pl.BlockSpec((1, S, HID), lambda b: (b, 0, 0)),
            pl.BlockSpec((1, S, S), lambda b: (b, 0, 0)),
            _rep((HID, H_GAT * HID)),
            _rep((1, H_GAT * HID)),
            _rep((HID, H_GAT * HID)),
            _rep((1, H_GAT * HID)),
            _rep((H_GAT, HID)),
            _rep((1, HID)),
        ],
        out_specs=pl.BlockSpec((1, S, HID), lambda b: (b, 0, 0)),
        out_shape=jax.ShapeDtypeStruct((B, S, HID), F32),
    )(ctx, adj, gat_wl, gat_bl[None, :], gat_wr, gat_br[None, :], gat_att,
      gat_bias[None, :])

    blwk_aa = jnp.kron(jnp.eye(NH_ATT, dtype=F32), aa_blw.T)
    blwk_ac = jnp.kron(jnp.eye(NH_ATT, dtype=F32), ac_blw.T)

    out3 = pl.pallas_call(
        _final_body,
        grid=(B,),
        in_specs=[
            pl.BlockSpec((1, S, HID), lambda b: (b, 0, 0)),
            pl.BlockSpec((1, S, HID), lambda b: (b, 0, 0)),
            pl.BlockSpec((1, A, HID), lambda b: (b, 0, 0)),
            pl.BlockSpec((1, S, 1), lambda b: (b, 0, 0)),
            pl.BlockSpec((1, S, 1), lambda b: (b, 0, 0)),
            pl.BlockSpec((1, 1, 1), lambda b: (b, 0, 0)),
            pl.BlockSpec((1, 1, 1), lambda b: (b, 0, 0)),
            _rep((HID, HID)), _rep((1, HID)), _rep((HID, HID)), _rep((1, HID)),
            _rep((HID, HID)), _rep((HID, HID)), _rep((1, HID)),
            _rep((HID, HID)), _rep((1, HID)), _rep((HID, HID)), _rep((1, HID)),
            _rep((HID, HID)), _rep((HID, HID)), _rep((1, HID)),
            _rep((3 * HID, POL)), _rep((1, POL)),
        ],
        out_specs=pl.BlockSpec((1, 1, POL), lambda b: (b, 0, 0)),
        out_shape=jax.ShapeDtypeStruct((B, 1, POL), F32),
    )(xi, ctx, asp, base[:, :, None], maskp[:, :, None],
      tl[:, :, None], al[:, :, None],
      aa_wk, aa_bk[None, :], aa_wq, aa_bq[None, :], blwk_aa, aa_pw,
      aa_pb[None, :],
      ac_wk, ac_bk[None, :], ac_wq, ac_bq[None, :], blwk_ac, ac_pw,
      ac_pb[None, :],
      fc_w, fc_b[None, :])

    return out3.reshape(B, POL)


# speedup vs baseline: 1.7465x; 1.7465x over previous
import functools

import jax
import jax.numpy as jnp
from jax import lax
from jax.experimental import pallas as pl
from jax.experimental.pallas import tpu as pltpu

B, S, A = 16, 64, 10
VOCAB, EMB, HID = 100000, 300, 64
NH_ATT = 32
HD_ATT = HID // NH_ATT
H_GAT = 16
POL = 3
F32 = jnp.float32


def _lrelu(x, slope):
    return jnp.where(x >= 0, x, x * slope)


_NROWS = S * B + A * B


def _gru_body(table_ref, idx_ref, ti_ref, ai_ref, li_ref, adj_ref,
              wihc_ref, whhc_ref, bihc_ref, bhhc_ref,
              wiha_ref, whha_ref, biha_ref, bhha_ref,
              ctx_out_ref, asp_out_ref, base_ref, maskp_ref, tl_ref, al_ref,
              xpc_ref, xpa_ref, rows_ref, dsem):
    def issue(j, c):
        ridx = idx_ref[j]
        pltpu.make_async_copy(table_ref.at[pl.ds(ridx, 1), :],
                              rows_ref.at[pl.ds(j, 1), :], dsem).start()
        return c

    lax.fori_loop(0, _NROWS, issue, 0)

    def drain(j, c):
        pltpu.make_async_copy(table_ref.at[pl.ds(0, 1), :],
                              rows_ref.at[pl.ds(j, 1), :], dsem).wait()
        return c

    lax.fori_loop(0, _NROWS, drain, 0)
    tl = jnp.sum((ti_ref[...] != 0).astype(F32), axis=1, keepdims=True) + 5.0
    al = jnp.sum((ai_ref[...] != 0).astype(F32), axis=1, keepdims=True)
    ll = jnp.sum((li_ref[...] != 0).astype(F32), axis=1, keepdims=True)
    tl_ref[...] = tl
    al_ref[...] = al

    j = lax.broadcasted_iota(jnp.int32, (B, S), 1).astype(F32)
    base = (j >= ll) & (j <= ll + al - 1.0)
    adj_sum = jnp.sum(adj_ref[...][:, :, S - 5:], axis=2)
    maskp = base | ((adj_sum != 0) & (j < S - 5))
    base_ref[...] = base.astype(F32)
    maskp_ref[...] = maskp.astype(F32)

    def run_gru(x2d, wih_t, whh_t, bih, bhh, T, lens, out_ref, xp_ref):
        xp = jnp.dot(x2d, wih_t, preferred_element_type=F32) + bih
        xp_ref[...] = xp.reshape(B, T, 3 * HID)

        def step(t, h):
            xpt = xp_ref[:, pl.ds(t, 1), :][:, 0, :]
            hh = jnp.dot(h, whh_t, preferred_element_type=F32)
            r = jax.nn.sigmoid(xpt[:, 0:HID] + hh[:, 0:HID] + bhh[:, 0:HID])
            z = jax.nn.sigmoid(xpt[:, HID:2 * HID] + hh[:, HID:2 * HID]
                               + bhh[:, HID:2 * HID])
            n = jnp.tanh(xpt[:, 2 * HID:] + r * (hh[:, 2 * HID:]
                                                 + bhh[:, 2 * HID:]))
            h2 = (1.0 - z) * n + z * h
            tmask = (lens > t.astype(F32)).astype(F32)
            out_ref[:, pl.ds(t, 1), :] = (h2 * tmask)[:, None, :]
            return h2

        lax.fori_loop(0, T, step, jnp.zeros((B, HID), F32))

    run_gru(rows_ref[0:S * B, :], wihc_ref[...], whhc_ref[...], bihc_ref[...],
            bhhc_ref[...], S, tl, ctx_out_ref, xpc_ref)
    run_gru(rows_ref[S * B:_NROWS, :], wiha_ref[...], whha_ref[...],
            biha_ref[...], bhha_ref[...], A, al, asp_out_ref, xpa_ref)


def _gat_body(x_ref, adj_ref, wl_ref, bl_ref, wr_ref, br_ref, att_ref,
              gbias_ref, out_ref):
    x = x_ref[0]
    adjb = adj_ref[0]
    xl = jnp.dot(x, wl_ref[...], preferred_element_type=F32) + bl_ref[...]
    xr = jnp.dot(x, wr_ref[...], preferred_element_type=F32) + br_ref[...]

    r_i = lax.broadcasted_iota(jnp.int32, (S, S), 0)
    c_i = lax.broadcasted_iota(jnp.int32, (S, S), 1)
    allowed = ((adjb != 0) & (r_i != c_i)) | (r_i == c_i)

    acc = jnp.zeros((S, HID), F32)
    for h in range(H_GAT):
        xlh = xl[:, h * HID:(h + 1) * HID]
        xrh = xr[:, h * HID:(h + 1) * HID]
        e = _lrelu(xlh[:, None, :] + xrh[None, :, :], 0.2)
        att3 = att_ref[h:h + 1, :][None]
        logits = jnp.sum(e * att3, axis=2)
        ml = jnp.where(allowed, logits, -1e30)
        amax = jnp.max(ml, axis=0, keepdims=True)
        p = jnp.where(allowed, jnp.exp(logits - amax), 0.0)
        denom = jnp.sum(p, axis=0, keepdims=True)
        alpha = p / (denom + 1e-16)
        acc = acc + lax.dot_general(alpha, xlh, (((0,), (0,)), ((), ())),
                                    preferred_element_type=F32)

    xi = _lrelu(acc * (1.0 / H_GAT) + gbias_ref[...], 0.01)
    out_ref[0] = xi


def _att_head(k, q, wk_ref, bk_ref, wq_ref, bq_ref, blwk_ref, pw_ref, pb_ref,
              epairs, edup):
    kx = jnp.dot(k, wk_ref[...], preferred_element_type=F32) + bk_ref[...]
    qx = jnp.dot(q, wq_ref[...], preferred_element_type=F32) + bq_ref[...]
    kxp = jnp.dot(kx, blwk_ref[...], preferred_element_type=F32)
    logits = jnp.dot(kxp * qx, epairs, preferred_element_type=F32)
    m = jnp.max(logits, axis=0, keepdims=True)
    sc = jnp.exp(logits - m)
    sc = sc / jnp.sum(sc, axis=0, keepdims=True)
    sd = jnp.dot(sc, edup, preferred_element_type=F32)
    outf = jnp.sum(kx * sd, axis=0, keepdims=True)
    return jnp.dot(outf, pw_ref[...], preferred_element_type=F32) + pb_ref[...]


def _final_body(xi_ref, ctx_ref, asp_ref, base_ref, maskp_ref, tl_ref, al_ref,
                aawk_ref, aabk_ref, aawq_ref, aabq_ref, aablwk_ref, aapw_ref,
                aapb_ref, acwk_ref, acbk_ref, acwq_ref, acbq_ref, acblwk_ref,
                acpw_ref, acpb_ref, fcw_ref, fcb_ref, out_ref):
    ip_r = lax.broadcasted_iota(jnp.int32, (HID, NH_ATT), 0)
    ip_c = lax.broadcasted_iota(jnp.int32, (HID, NH_ATT), 1)
    epairs = (ip_r // HD_ATT == ip_c).astype(F32)
    id_r = lax.broadcasted_iota(jnp.int32, (NH_ATT, HID), 0)
    id_c = lax.broadcasted_iota(jnp.int32, (NH_ATT, HID), 1)
    edup = (id_c // HD_ATT == id_r).astype(F32)

    xi = xi_ref[0]
    ctx = ctx_ref[0]
    asp = asp_ref[0]
    x = xi * base_ref[0]
    cm = ctx * maskp_ref[0]

    am = lax.dot_general(x, cm, (((1,), (1,)), ((), ())),
                         preferred_element_type=F32)
    s = jnp.sum(am, axis=0, keepdims=True)
    s = s - jnp.max(s, axis=1, keepdims=True)
    es = jnp.exp(s)
    alpha = es / jnp.sum(es, axis=1, keepdims=True)
    gta = jnp.dot(alpha, cm, preferred_element_type=F32)

    al = al_ref[0]
    tl = tl_ref[0]
    asp_pool = jnp.sum(asp, axis=0, keepdims=True) / al
    ctx_pool = jnp.sum(ctx, axis=0, keepdims=True) / tl

    af = _att_head(asp, ctx_pool, aawk_ref, aabk_ref, aawq_ref, aabq_ref,
                   aablwk_ref, aapw_ref, aapb_ref, epairs, edup)
    cf = _att_head(ctx, asp_pool, acwk_ref, acbk_ref, acwq_ref, acbq_ref,
                   acblwk_ref, acpw_ref, acpb_ref, epairs, edup)

    fcw = fcw_ref[...]
    out = (jnp.dot(af, fcw[0:HID], preferred_element_type=F32)
           + jnp.dot(cf, fcw[HID:2 * HID], preferred_element_type=F32)
           + jnp.dot(gta, fcw[2 * HID:], preferred_element_type=F32)
           + fcb_ref[...])
    out_ref[0] = out


def _rep(shape):
    nd = len(shape)
    return pl.BlockSpec(shape, lambda b: (0,) * nd)


@jax.jit
def kernel(text_indices, aspect_indices, left_indices, adj, embed_table,
           gru_ctx_wih, gru_ctx_whh, gru_ctx_bih, gru_ctx_bhh,
           gru_asp_wih, gru_asp_whh, gru_asp_bih, gru_asp_bhh,
           aa_wk, aa_bk, aa_wq, aa_bq, aa_blw, aa_pw, aa_pb,
           ac_wk, ac_bk, ac_wq, ac_bq, ac_blw, ac_pw, ac_pb,
           gat_wl, gat_bl, gat_wr, gat_br, gat_att, gat_bias, fc_w, fc_b):
    idx_all = jnp.concatenate([
        text_indices.reshape(-1), aspect_indices.reshape(-1)])

    vspec = pl.BlockSpec(memory_space=pltpu.MemorySpace.VMEM)

    ctx, asp, base, maskp, tl, al = pl.pallas_call(
        _gru_body,
        in_specs=[pl.BlockSpec(memory_space=pltpu.MemorySpace.HBM),
                  pl.BlockSpec(memory_space=pltpu.MemorySpace.SMEM)]
        + [vspec] * 12,
        out_shape=[
            jax.ShapeDtypeStruct((B, S, HID), F32),
            jax.ShapeDtypeStruct((B, A, HID), F32),
            jax.ShapeDtypeStruct((B, S), F32),
            jax.ShapeDtypeStruct((B, S), F32),
            jax.ShapeDtypeStruct((B, 1), F32),
            jax.ShapeDtypeStruct((B, 1), F32),
        ],
        scratch_shapes=[
            pltpu.VMEM((B, S, 3 * HID), F32),
            pltpu.VMEM((B, A, 3 * HID), F32),
            pltpu.VMEM((_NROWS, EMB), F32),
            pltpu.SemaphoreType.DMA,
        ],
    )(embed_table, idx_all, text_indices, aspect_indices, left_indices, adj,
      gru_ctx_wih.T, gru_ctx_whh.T, gru_ctx_bih[None, :], gru_ctx_bhh[None, :],
      gru_asp_wih.T, gru_asp_whh.T, gru_asp_bih[None, :], gru_asp_bhh[None, :])

    xi = pl.pallas_call(
        _gat_body,
        grid=(B,),
        in_specs=[
            pl.BlockSpec((1, S, HID), lambda b: (b, 0, 0)),
            pl.BlockSpec((1, S, S), lambda b: (b, 0, 0)),
            _rep((HID, H_GAT * HID)),
            _rep((1, H_GAT * HID)),
            _rep((HID, H_GAT * HID)),
            _rep((1, H_GAT * HID)),
            _rep((H_GAT, HID)),
            _rep((1, HID)),
        ],
        out_specs=pl.BlockSpec((1, S, HID), lambda b: (b, 0, 0)),
        out_shape=jax.ShapeDtypeStruct((B, S, HID), F32),
    )(ctx, adj, gat_wl, gat_bl[None, :], gat_wr, gat_br[None, :], gat_att,
      gat_bias[None, :])

    blwk_aa = jnp.kron(jnp.eye(NH_ATT, dtype=F32), aa_blw.T)
    blwk_ac = jnp.kron(jnp.eye(NH_ATT, dtype=F32), ac_blw.T)

    out3 = pl.pallas_call(
        _final_body,
        grid=(B,),
        in_specs=[
            pl.BlockSpec((1, S, HID), lambda b: (b, 0, 0)),
            pl.BlockSpec((1, S, HID), lambda b: (b, 0, 0)),
            pl.BlockSpec((1, A, HID), lambda b: (b, 0, 0)),
            pl.BlockSpec((1, S, 1), lambda b: (b, 0, 0)),
            pl.BlockSpec((1, S, 1), lambda b: (b, 0, 0)),
            pl.BlockSpec((1, 1, 1), lambda b: (b, 0, 0)),
            pl.BlockSpec((1, 1, 1), lambda b: (b, 0, 0)),
            _rep((HID, HID)), _rep((1, HID)), _rep((HID, HID)), _rep((1, HID)),
            _rep((HID, HID)), _rep((HID, HID)), _rep((1, HID)),
            _rep((HID, HID)), _rep((1, HID)), _rep((HID, HID)), _rep((1, HID)),
            _rep((HID, HID)), _rep((HID, HID)), _rep((1, HID)),
            _rep((3 * HID, POL)), _rep((1, POL)),
        ],
        out_specs=pl.BlockSpec((1, 1, POL), lambda b: (b, 0, 0)),
        out_shape=jax.ShapeDtypeStruct((B, 1, POL), F32),
    )(xi, ctx, asp, base[:, :, None], maskp[:, :, None],
      tl[:, :, None], al[:, :, None],
      aa_wk, aa_bk[None, :], aa_wq, aa_bq[None, :], blwk_aa, aa_pw,
      aa_pb[None, :],
      ac_wk, ac_bk[None, :], ac_wq, ac_bq[None, :], blwk_ac, ac_pw,
      ac_pb[None, :],
      fc_w, fc_b[None, :])

    return out3.reshape(B, POL)


# speedup vs baseline: 1.7500x; 1.0020x over previous
import functools

import jax
import jax.numpy as jnp
from jax import lax
from jax.experimental import pallas as pl
from jax.experimental.pallas import tpu as pltpu

B, S, A = 16, 64, 10
VOCAB, EMB, HID = 100000, 300, 64
NH_ATT = 32
HD_ATT = HID // NH_ATT
H_GAT = 16
POL = 3
F32 = jnp.float32


def _lrelu(x, slope):
    return jnp.where(x >= 0, x, x * slope)


_NROWS = S * B + A * B


def _gru_body(table_ref, idx_ref, ti_ref, ai_ref, li_ref, adj_ref,
              wihc_ref, whhc_ref, bihc_ref, bhhc_ref,
              wiha_ref, whha_ref, biha_ref, bhha_ref,
              ctx_out_ref, asp_out_ref, base_ref, maskp_ref, tl_ref, al_ref,
              xpc_ref, xpa_ref, rows_ref, dsem):
    def issue(j, c):
        ridx = idx_ref[j]
        pltpu.make_async_copy(table_ref.at[pl.ds(ridx, 1), :],
                              rows_ref.at[pl.ds(j, 1), :], dsem).start()
        return c

    lax.fori_loop(0, _NROWS, issue, 0)

    def drain(j, c):
        pltpu.make_async_copy(table_ref.at[pl.ds(0, 1), :],
                              rows_ref.at[pl.ds(j, 1), :], dsem).wait()
        return c

    lax.fori_loop(0, _NROWS, drain, 0)
    tl = jnp.sum((ti_ref[...] != 0).astype(F32), axis=1, keepdims=True) + 5.0
    al = jnp.sum((ai_ref[...] != 0).astype(F32), axis=1, keepdims=True)
    ll = jnp.sum((li_ref[...] != 0).astype(F32), axis=1, keepdims=True)
    tl_ref[...] = tl
    al_ref[...] = al

    j = lax.broadcasted_iota(jnp.int32, (B, S), 1).astype(F32)
    base = (j >= ll) & (j <= ll + al - 1.0)
    adj_sum = jnp.sum(adj_ref[...][:, :, S - 5:], axis=2)
    maskp = base | ((adj_sum != 0) & (j < S - 5))
    base_ref[...] = base.astype(F32)
    maskp_ref[...] = maskp.astype(F32)

    def run_gru(x2d, wih_t, whh_t, bih, bhh, T, lens, out_ref, xp_ref):
        xp = jnp.dot(x2d, wih_t, preferred_element_type=F32) + bih
        xp_ref[...] = xp.reshape(B, T, 3 * HID)

        def step(t, h):
            xpt = xp_ref[:, pl.ds(t, 1), :][:, 0, :]
            hh = jnp.dot(h, whh_t, preferred_element_type=F32)
            r = jax.nn.sigmoid(xpt[:, 0:HID] + hh[:, 0:HID] + bhh[:, 0:HID])
            z = jax.nn.sigmoid(xpt[:, HID:2 * HID] + hh[:, HID:2 * HID]
                               + bhh[:, HID:2 * HID])
            n = jnp.tanh(xpt[:, 2 * HID:] + r * (hh[:, 2 * HID:]
                                                 + bhh[:, 2 * HID:]))
            h2 = (1.0 - z) * n + z * h
            tmask = (lens > t.astype(F32)).astype(F32)
            out_ref[:, pl.ds(t, 1), :] = (h2 * tmask)[:, None, :]
            return h2

        lax.fori_loop(0, T, step, jnp.zeros((B, HID), F32))

    run_gru(rows_ref[0:S * B, :], wihc_ref[...], whhc_ref[...], bihc_ref[...],
            bhhc_ref[...], S, tl, ctx_out_ref, xpc_ref)
    run_gru(rows_ref[S * B:_NROWS, :], wiha_ref[...], whha_ref[...],
            biha_ref[...], bhha_ref[...], A, al, asp_out_ref, xpa_ref)


def _gat_body(x_ref, adj_ref, wl_ref, bl_ref, wr_ref, br_ref, att_ref,
              gbias_ref, out_ref):
    x = x_ref[0]
    adjb = adj_ref[0]
    xl = jnp.dot(x, wl_ref[...], preferred_element_type=F32) + bl_ref[...]
    xr = jnp.dot(x, wr_ref[...], preferred_element_type=F32) + br_ref[...]

    r_i = lax.broadcasted_iota(jnp.int32, (S, S), 0)
    c_i = lax.broadcasted_iota(jnp.int32, (S, S), 1)
    allowed = ((adjb != 0) & (r_i != c_i)) | (r_i == c_i)

    acc = jnp.zeros((S, HID), F32)
    for h in range(H_GAT):
        xlh = xl[:, h * HID:(h + 1) * HID]
        xrh = xr[:, h * HID:(h + 1) * HID]
        e = _lrelu(xlh[:, None, :] + xrh[None, :, :], 0.2)
        logits = lax.dot_general(
            e.reshape(S * S, HID), att_ref[h:h + 1, :],
            (((1,), (1,)), ((), ())),
            preferred_element_type=F32).reshape(S, S)
        ml = jnp.where(allowed, logits, -1e30)
        amax = jnp.max(ml, axis=0, keepdims=True)
        p = jnp.where(allowed, jnp.exp(logits - amax), 0.0)
        denom = jnp.sum(p, axis=0, keepdims=True)
        alpha = p / (denom + 1e-16)
        acc = acc + lax.dot_general(alpha, xlh, (((0,), (0,)), ((), ())),
                                    preferred_element_type=F32)

    xi = _lrelu(acc * (1.0 / H_GAT) + gbias_ref[...], 0.01)
    out_ref[0] = xi


def _att_head(k, q, wk_ref, bk_ref, wq_ref, bq_ref, blwk_ref, pw_ref, pb_ref,
              epairs, edup):
    kx = jnp.dot(k, wk_ref[...], preferred_element_type=F32) + bk_ref[...]
    qx = jnp.dot(q, wq_ref[...], preferred_element_type=F32) + bq_ref[...]
    kxp = jnp.dot(kx, blwk_ref[...], preferred_element_type=F32)
    logits = jnp.dot(kxp * qx, epairs, preferred_element_type=F32)
    m = jnp.max(logits, axis=0, keepdims=True)
    sc = jnp.exp(logits - m)
    sc = sc / jnp.sum(sc, axis=0, keepdims=True)
    sd = jnp.dot(sc, edup, preferred_element_type=F32)
    outf = jnp.sum(kx * sd, axis=0, keepdims=True)
    return jnp.dot(outf, pw_ref[...], preferred_element_type=F32) + pb_ref[...]


def _final_body(xi_ref, ctx_ref, asp_ref, base_ref, maskp_ref, tl_ref, al_ref,
                aawk_ref, aabk_ref, aawq_ref, aabq_ref, aablwk_ref, aapw_ref,
                aapb_ref, acwk_ref, acbk_ref, acwq_ref, acbq_ref, acblwk_ref,
                acpw_ref, acpb_ref, fcw_ref, fcb_ref, out_ref):
    ip_r = lax.broadcasted_iota(jnp.int32, (HID, NH_ATT), 0)
    ip_c = lax.broadcasted_iota(jnp.int32, (HID, NH_ATT), 1)
    epairs = (ip_r // HD_ATT == ip_c).astype(F32)
    id_r = lax.broadcasted_iota(jnp.int32, (NH_ATT, HID), 0)
    id_c = lax.broadcasted_iota(jnp.int32, (NH_ATT, HID), 1)
    edup = (id_c // HD_ATT == id_r).astype(F32)

    xi = xi_ref[0]
    ctx = ctx_ref[0]
    asp = asp_ref[0]
    x = xi * base_ref[0]
    cm = ctx * maskp_ref[0]

    am = lax.dot_general(x, cm, (((1,), (1,)), ((), ())),
                         preferred_element_type=F32)
    s = jnp.sum(am, axis=0, keepdims=True)
    s = s - jnp.max(s, axis=1, keepdims=True)
    es = jnp.exp(s)
    alpha = es / jnp.sum(es, axis=1, keepdims=True)
    gta = jnp.dot(alpha, cm, preferred_element_type=F32)

    al = al_ref[0]
    tl = tl_ref[0]
    asp_pool = jnp.sum(asp, axis=0, keepdims=True) / al
    ctx_pool = jnp.sum(ctx, axis=0, keepdims=True) / tl

    af = _att_head(asp, ctx_pool, aawk_ref, aabk_ref, aawq_ref, aabq_ref,
                   aablwk_ref, aapw_ref, aapb_ref, epairs, edup)
    cf = _att_head(ctx, asp_pool, acwk_ref, acbk_ref, acwq_ref, acbq_ref,
                   acblwk_ref, acpw_ref, acpb_ref, epairs, edup)

    fcw = fcw_ref[...]
    out = (jnp.dot(af, fcw[0:HID], preferred_element_type=F32)
           + jnp.dot(cf, fcw[HID:2 * HID], preferred_element_type=F32)
           + jnp.dot(gta, fcw[2 * HID:], preferred_element_type=F32)
           + fcb_ref[...])
    out_ref[0] = out


def _rep(shape):
    nd = len(shape)
    return pl.BlockSpec(shape, lambda b: (0,) * nd)


@jax.jit
def kernel(text_indices, aspect_indices, left_indices, adj, embed_table,
           gru_ctx_wih, gru_ctx_whh, gru_ctx_bih, gru_ctx_bhh,
           gru_asp_wih, gru_asp_whh, gru_asp_bih, gru_asp_bhh,
           aa_wk, aa_bk, aa_wq, aa_bq, aa_blw, aa_pw, aa_pb,
           ac_wk, ac_bk, ac_wq, ac_bq, ac_blw, ac_pw, ac_pb,
           gat_wl, gat_bl, gat_wr, gat_br, gat_att, gat_bias, fc_w, fc_b):
    idx_all = jnp.concatenate([
        text_indices.reshape(-1), aspect_indices.reshape(-1)])

    vspec = pl.BlockSpec(memory_space=pltpu.MemorySpace.VMEM)

    ctx, asp, base, maskp, tl, al = pl.pallas_call(
        _gru_body,
        in_specs=[pl.BlockSpec(memory_space=pltpu.MemorySpace.HBM),
                  pl.BlockSpec(memory_space=pltpu.MemorySpace.SMEM)]
        + [vspec] * 12,
        out_shape=[
            jax.ShapeDtypeStruct((B, S, HID), F32),
            jax.ShapeDtypeStruct((B, A, HID), F32),
            jax.ShapeDtypeStruct((B, S), F32),
            jax.ShapeDtypeStruct((B, S), F32),
            jax.ShapeDtypeStruct((B, 1), F32),
            jax.ShapeDtypeStruct((B, 1), F32),
        ],
        scratch_shapes=[
            pltpu.VMEM((B, S, 3 * HID), F32),
            pltpu.VMEM((B, A, 3 * HID), F32),
            pltpu.VMEM((_NROWS, EMB), F32),
            pltpu.SemaphoreType.DMA,
        ],
    )(embed_table, idx_all, text_indices, aspect_indices, left_indices, adj,
      gru_ctx_wih.T, gru_ctx_whh.T, gru_ctx_bih[None, :], gru_ctx_bhh[None, :],
      gru_asp_wih.T, gru_asp_whh.T, gru_asp_bih[None, :], gru_asp_bhh[None, :])

    xi = pl.pallas_call(
        _gat_body,
        grid=(B,),
        in_specs=[
            pl.BlockSpec((1, S, HID), lambda b: (b, 0, 0)),
            pl.BlockSpec((1, S, S), lambda b: (b, 0, 0)),
            _rep((HID, H_GAT * HID)),
            _rep((1, H_GAT * HID)),
            _rep((HID, H_GAT * HID)),
            _rep((1, H_GAT * HID)),
            _rep((H_GAT, HID)),
            _rep((1, HID)),
        ],
        out_specs=pl.BlockSpec((1, S, HID), lambda b: (b, 0, 0)),
        out_shape=jax.ShapeDtypeStruct((B, S, HID), F32),
    )(ctx, adj, gat_wl, gat_bl[None, :], gat_wr, gat_br[None, :], gat_att,
      gat_bias[None, :])

    blwk_aa = jnp.kron(jnp.eye(NH_ATT, dtype=F32), aa_blw.T)
    blwk_ac = jnp.kron(jnp.eye(NH_ATT, dtype=F32), ac_blw.T)

    out3 = pl.pallas_call(
        _final_body,
        grid=(B,),
        in_specs=[
            pl.BlockSpec((1, S, HID), lambda b: (b, 0, 0)),
            pl.BlockSpec((1, S, HID), lambda b: (b, 0, 0)),
            pl.BlockSpec((1, A, HID), lambda b: (b, 0, 0)),
            pl.BlockSpec((1, S, 1), lambda b: (b, 0, 0)),
            pl.BlockSpec((1, S, 1), lambda b: (b, 0, 0)),
            pl.BlockSpec((1, 1, 1), lambda b: (b, 0, 0)),
            pl.BlockSpec((1, 1, 1), lambda b: (b, 0, 0)),
            _rep((HID, HID)), _rep((1, HID)), _rep((HID, HID)), _rep((1, HID)),
            _rep((HID, HID)), _rep((HID, HID)), _rep((1, HID)),
            _rep((HID, HID)), _rep((1, HID)), _rep((HID, HID)), _rep((1, HID)),
            _rep((HID, HID)), _rep((HID, HID)), _rep((1, HID)),
            _rep((3 * HID, POL)), _rep((1, POL)),
        ],
        out_specs=pl.BlockSpec((1, 1, POL), lambda b: (b, 0, 0)),
        out_shape=jax.ShapeDtypeStruct((B, 1, POL), F32),
    )(xi, ctx, asp, base[:, :, None], maskp[:, :, None],
      tl[:, :, None], al[:, :, None],
      aa_wk, aa_bk[None, :], aa_wq, aa_bq[None, :], blwk_aa, aa_pw,
      aa_pb[None, :],
      ac_wk, ac_bk[None, :], ac_wq, ac_bq[None, :], blwk_ac, ac_pw,
      ac_pb[None, :],
      fc_w, fc_b[None, :])

    return out3.reshape(B, POL)
